# Initial kernel scaffold; baseline (speedup 1.0000x reference)
import jax
import jax.numpy as jnp
from jax.experimental import pallas as pl


def kernel(mlp_0, mlp_1, W_enc_0, b_enc_0, W_dec_0, b_dec_0, W_enc_1, b_enc_1, W_dec_1, b_dec_1):
    raise NotImplementedError("write your pallas kernel here")


# speedup vs baseline: 8.2036x; 8.2036x over previous
import jax
import jax.numpy as jnp
from jax import lax
from jax.experimental import pallas as pl

K = 64


def _encode_body(x_ref, w_ref, be_ref, bd_ref, out_ref):
    x = x_ref[...]
    w = w_ref[...]
    be = be_ref[...]
    bd = bd_ref[...]
    badj = be - lax.dot_general(bd.astype(jnp.bfloat16), w.astype(jnp.bfloat16),
                                (((1,), (1,)), ((), ())),
                                preferred_element_type=jnp.float32)
    y = lax.dot_general(x.astype(jnp.bfloat16), w.astype(jnp.bfloat16),
                        (((1,), (1,)), ((), ())),
                        preferred_element_type=jnp.float32)
    out_ref[...] = jnp.maximum(y + badj, 0.0)


def _encode(x, W_enc, b_enc, b_dec, bf):
    n, d = x.shape
    f = W_enc.shape[0]
    return pl.pallas_call(
        _encode_body,
        grid=(f // bf,),
        in_specs=[
            pl.BlockSpec((n, d), lambda i: (0, 0)),
            pl.BlockSpec((bf, d), lambda i: (i, 0)),
            pl.BlockSpec((1, bf), lambda i: (0, i)),
            pl.BlockSpec((1, d), lambda i: (0, 0)),
        ],
        out_specs=pl.BlockSpec((n, bf), lambda i: (0, i)),
        out_shape=jax.ShapeDtypeStruct((n, f), jnp.float32),
    )(x, W_enc, b_enc.reshape(1, f), b_dec.reshape(1, d))


def _topk_body(pre_ref, tb_ref, ti_ref):
    v = pre_ref[...]
    f = v.shape[1]
    bits = lax.bitcast_convert_type(v, jnp.int32)
    rowmax = jnp.max(bits, axis=1, keepdims=True)
    lo = jnp.zeros_like(rowmax)
    hi = rowmax
    def body(_, st):
        blo, bhi = st
        mid = blo + lax.div(bhi - blo, 2)
        cnt = jnp.sum((bits > mid).astype(jnp.int32), axis=1, keepdims=True)
        p = cnt < K
        return jnp.where(p, blo, mid + 1), jnp.where(p, mid, bhi)

    _, tbits = lax.fori_loop(0, 31, body, (lo, hi))

    gt = bits > tbits
    eq = bits == tbits
    c_gt = jnp.sum(gt.astype(jnp.int32), axis=1, keepdims=True)
    c_eq = jnp.sum(eq.astype(jnp.int32), axis=1, keepdims=True)
    need = K - c_gt
    idx = lax.broadcasted_iota(jnp.int32, v.shape, 1)
    solved = c_eq == need
    lo2 = jnp.where(solved, f - 1, 0)
    hi2 = jnp.full_like(lo2, f - 1)

    def body2(_, st):
        blo, bhi = st
        mid = blo + lax.div(bhi - blo, 2)
        cnt = jnp.sum((eq & (idx <= mid)).astype(jnp.int32),
                      axis=1, keepdims=True)
        p = cnt >= need
        return jnp.where(p, blo, mid + 1), jnp.where(p, mid, bhi)

    _, tidx = lax.fori_loop(0, 15, body2, (lo2, hi2))

    tb_ref[...] = jnp.broadcast_to(tbits, tb_ref.shape)
    ti_ref[...] = jnp.broadcast_to(tidx, ti_ref.shape)


def _topk_thresholds(pre, bn):
    n, f = pre.shape
    return pl.pallas_call(
        _topk_body,
        grid=(n // bn,),
        in_specs=[pl.BlockSpec((bn, f), lambda i: (i, 0))],
        out_specs=[pl.BlockSpec((bn, 128), lambda i: (i, 0)),
                   pl.BlockSpec((bn, 128), lambda i: (i, 0))],
        out_shape=[jax.ShapeDtypeStruct((n, 128), jnp.int32),
                   jax.ShapeDtypeStruct((n, 128), jnp.int32)],
    )(pre)


def _decode_body(pre_ref, w_ref, tb_ref, ti_ref, bd_ref, out_ref, *, bf):
    i = pl.program_id(0)
    v = pre_ref[...]
    bits = lax.bitcast_convert_type(v, jnp.int32)
    tb = tb_ref[:, 0:1]
    ti = ti_ref[:, 0:1]
    gidx = lax.broadcasted_iota(jnp.int32, v.shape, 1) + i * bf
    mask = (bits > tb) | ((bits == tb) & (gidx <= ti))
    feat = jnp.where(mask, v, 0.0)
    w = w_ref[...]
    part = lax.dot_general(feat.astype(jnp.bfloat16), w.astype(jnp.bfloat16),
                           (((1,), (1,)), ((), ())),
                           preferred_element_type=jnp.float32)

    @pl.when(i == 0)
    def _():
        out_ref[...] = bd_ref[...] + part

    @pl.when(i > 0)
    def _():
        out_ref[...] += part


def _decode(pre, W_dec, tb, ti, b_dec, bf):
    import functools
    n, f = pre.shape
    d = W_dec.shape[0]
    return pl.pallas_call(
        functools.partial(_decode_body, bf=bf),
        grid=(f // bf,),
        in_specs=[
            pl.BlockSpec((n, bf), lambda i: (0, i)),
            pl.BlockSpec((d, bf), lambda i: (0, i)),
            pl.BlockSpec((n, 128), lambda i: (0, 0)),
            pl.BlockSpec((n, 128), lambda i: (0, 0)),
            pl.BlockSpec((1, d), lambda i: (0, 0)),
        ],
        out_specs=pl.BlockSpec((n, d), lambda i: (0, 0)),
        out_shape=jax.ShapeDtypeStruct((n, d), jnp.float32),
    )(pre, W_dec, tb, ti, b_dec.reshape(1, d))


def _ae_forward(x, W_enc, b_enc, W_dec, b_dec, bf_enc=512, bn_top=128,
                bf_dec=512):
    pre = _encode(x, W_enc, b_enc, b_dec, bf_enc)
    tb, ti = _topk_thresholds(pre, bn_top)
    return _decode(pre, W_dec, tb, ti, b_dec, bf_dec)


def kernel(mlp_0, mlp_1, W_enc_0, b_enc_0, W_dec_0, b_dec_0,
           W_enc_1, b_enc_1, W_dec_1, b_dec_1):
    r0 = _ae_forward(mlp_0, W_enc_0, b_enc_0, W_dec_0, b_dec_0)
    r1 = _ae_forward(mlp_1, W_enc_1, b_enc_1, W_dec_1, b_dec_1)
    return (r0, r1)
